# Initial kernel scaffold; baseline (speedup 1.0000x reference)
import jax
import jax.numpy as jnp
from jax.experimental import pallas as pl


def kernel(padded_trajs, trajs_len, edge_index, edge_type, loc_emb, traj_emb, W_ih, W_hh, b_ih, b_hh, r1_Wrel, r1_Wroot, r1_b, r2_Wrel, r2_Wroot, r2_b, W_pred, b_pred):
    raise NotImplementedError("write your pallas kernel here")


# speedup vs baseline: 3.2758x; 3.2758x over previous
import functools

import jax
import jax.numpy as jnp
from jax import lax
from jax.experimental import pallas as pl
from jax.experimental.pallas import tpu as pltpu
from jax.experimental.pallas import tpu_sc as plsc

N_T = 10000
NP = 10240
NA = 10112
H = 128
L = 20
E = 160000
NW = 32
CH = 128
TOK = NP * L
GCH = TOK // NW // CH
EP_T = 10112
EPAD = 16 * EP_T
ECH = EP_T // CH
RPT = NA // 16
TRASH = N_T
BB = 1024
BD = 1000
F32 = jnp.float32


def _mesh():
    return plsc.VectorSubcoreMesh(core_axis_name="c", subcore_axis_name="s")


def _sc_gather(table, idx3):
    @functools.partial(
        pl.kernel,
        mesh=_mesh(),
        out_type=jax.ShapeDtypeStruct((TOK, H), F32),
        scratch_types=[
            pltpu.VMEM((GCH, CH), jnp.int32),
            pltpu.VMEM((CH, H), F32),
            pltpu.SemaphoreType.DMA,
        ],
    )
    def k(table_hbm, idx_hbm, out_hbm, idx_v, rows_v, sem):
        wid = lax.axis_index("s") * 2 + lax.axis_index("c")
        pltpu.sync_copy(idx_hbm.at[wid], idx_v)
        base = wid * (GCH * CH)

        def body(i, _):
            pltpu.async_copy(table_hbm.at[idx_v.at[i]], rows_v, sem).wait()
            pltpu.sync_copy(rows_v, out_hbm.at[pl.ds(base + i * CH, CH)])
            return 0

        lax.fori_loop(0, GCH, body, 0)

    return k(table, idx3)


def _sc_segsum(x, enc3, z128):
    @functools.partial(
        pl.kernel,
        mesh=_mesh(),
        out_type=jax.ShapeDtypeStruct((2, NA, H), F32),
        scratch_types=[
            pltpu.VMEM((ECH, CH), jnp.int32),
            pltpu.VMEM((ECH, CH), jnp.int32),
            pltpu.VMEM((CH, H), F32),
            pltpu.SemaphoreType.DMA,
            pltpu.VMEM_SHARED((NA, H), F32),
        ],
    )
    def k(x_hbm, enc_hbm, z_hbm, agg_hbm, enc_v, dst_v, rows_v, sem, acc):
        ci = lax.axis_index("c")
        sid = lax.axis_index("s")
        pltpu.sync_copy(z_hbm, acc.at[pl.ds(sid * RPT, RPT)])
        pltpu.sync_copy(enc_hbm.at[sid], enc_v)

        def fix_row(r, _):
            def fix16(j, _):
                o = j * 16
                e16 = enc_v[r, pl.ds(o, 16)]
                d16 = (e16 >> 14) & 16383
                t16 = e16 >> 28
                dst_v[r, pl.ds(o, 16)] = jnp.where(t16 == ci, d16, TRASH)
                enc_v[r, pl.ds(o, 16)] = e16 & 16383
                return 0

            lax.fori_loop(0, CH // 16, fix16, 0)
            return 0

        lax.fori_loop(0, ECH, fix_row, 0)
        plsc.subcore_barrier()

        def chunk(i, _):
            pltpu.async_copy(x_hbm.at[enc_v.at[i]], rows_v, sem).wait()
            pltpu.sync_copy(rows_v, acc.at[dst_v.at[i]], add=True)
            return 0

        lax.fori_loop(0, ECH, chunk, 0)
        plsc.subcore_barrier()
        sl = pl.ds(sid * RPT, RPT)
        pltpu.sync_copy(acc.at[sl], agg_hbm.at[ci, sl])

    return k(x, enc3, z128)


def _sc_counts(enc3, z128, o128):
    @functools.partial(
        pl.kernel,
        mesh=_mesh(),
        out_type=jax.ShapeDtypeStruct((2, NA, H), F32),
        scratch_types=[
            pltpu.VMEM((ECH, CH), jnp.int32),
            pltpu.VMEM((ECH, CH), jnp.int32),
            pltpu.VMEM((CH, H), F32),
            pltpu.SemaphoreType.DMA,
            pltpu.VMEM_SHARED((NA, H), F32),
        ],
    )
    def k(enc_hbm, z_hbm, o_hbm, cnt_hbm, enc_v, dst_v, ones_v, sem, cacc):
        ci = lax.axis_index("c")
        sid = lax.axis_index("s")
        pltpu.sync_copy(z_hbm, cacc.at[pl.ds(sid * RPT, RPT)])
        pltpu.sync_copy(o_hbm, ones_v)
        pltpu.sync_copy(enc_hbm.at[sid], enc_v)

        def fix_row(r, _):
            def fix16(j, _):
                o = j * 16
                e16 = enc_v[r, pl.ds(o, 16)]
                d16 = (e16 >> 14) & 16383
                t16 = e16 >> 28
                dst_v[r, pl.ds(o, 16)] = jnp.where(t16 == ci, d16, TRASH)
                return 0

            lax.fori_loop(0, CH // 16, fix16, 0)
            return 0

        lax.fori_loop(0, ECH, fix_row, 0)
        plsc.subcore_barrier()

        def chunk(i, _):
            pltpu.sync_copy(ones_v, cacc.at[dst_v.at[i]], add=True)
            return 0

        lax.fori_loop(0, ECH, chunk, 0)
        plsc.subcore_barrier()
        sl = pl.ds(sid * RPT, RPT)
        pltpu.sync_copy(cacc.at[sl], cnt_hbm.at[ci, sl])

    return k(enc3, z128, o128)


def _lstm(x, lens_b, wih_t, whh_t, bias2):
    def body(x_ref, len_ref, wi_ref, wh_ref, b_ref, out_ref):
        h = jnp.zeros((BB, H), F32)
        c = jnp.zeros((BB, H), F32)
        hl = jnp.zeros((BB, H), F32)
        lens = len_ref[:]
        wi = wi_ref[:]
        wh = wh_ref[:]
        b = b_ref[:]
        for t in range(L):
            xt = x_ref[:, t * H:(t + 1) * H]
            g = (jnp.dot(xt, wi, preferred_element_type=F32)
                 + jnp.dot(h, wh, preferred_element_type=F32) + b)
            i_g = jax.nn.sigmoid(g[:, 0:H])
            f_g = jax.nn.sigmoid(g[:, H:2 * H])
            g_g = jnp.tanh(g[:, 2 * H:3 * H])
            o_g = jax.nn.sigmoid(g[:, 3 * H:4 * H])
            c = f_g * c + i_g * g_g
            h = o_g * jnp.tanh(c)
            hl = jnp.where(lens == t + 1, h, hl)
        out_ref[:] = hl

    return pl.pallas_call(
        body,
        grid=(NP // BB,),
        in_specs=[
            pl.BlockSpec((BB, L * H), lambda i: (i, 0)),
            pl.BlockSpec((BB, H), lambda i: (i, 0)),
            pl.BlockSpec((H, 4 * H), lambda i: (0, 0)),
            pl.BlockSpec((H, 4 * H), lambda i: (0, 0)),
            pl.BlockSpec((1, 4 * H), lambda i: (0, 0)),
        ],
        out_specs=pl.BlockSpec((BB, H), lambda i: (i, 0)),
        out_shape=jax.ShapeDtypeStruct((NP, H), F32),
    )(x, lens_b, wih_t, whh_t, bias2)


def _dense1(a0, a1, c0, c1, x, w0, w1, wroot, bias):
    def body(a0r, a1r, c0r, c1r, xr, w0r, w1r, wrr, br, outr):
        m0 = a0r[:] / jnp.maximum(c0r[:, 0:1], 1.0)
        m1 = a1r[:] / jnp.maximum(c1r[:, 0:1], 1.0)
        outr[:] = (jnp.dot(m0, w0r[:], preferred_element_type=F32)
                   + jnp.dot(m1, w1r[:], preferred_element_type=F32)
                   + jnp.dot(xr[:], wrr[:], preferred_element_type=F32)
                   + br[:])

    return pl.pallas_call(
        body,
        grid=(N_T // BD,),
        in_specs=[
            pl.BlockSpec((BD, H), lambda i: (i, 0)),
            pl.BlockSpec((BD, H), lambda i: (i, 0)),
            pl.BlockSpec((BD, H), lambda i: (i, 0)),
            pl.BlockSpec((BD, H), lambda i: (i, 0)),
            pl.BlockSpec((BD, H), lambda i: (i, 0)),
            pl.BlockSpec((H, H), lambda i: (0, 0)),
            pl.BlockSpec((H, H), lambda i: (0, 0)),
            pl.BlockSpec((H, H), lambda i: (0, 0)),
            pl.BlockSpec((1, H), lambda i: (0, 0)),
        ],
        out_specs=pl.BlockSpec((BD, H), lambda i: (i, 0)),
        out_shape=jax.ShapeDtypeStruct((N_T, H), F32),
    )(a0, a1, c0, c1, x, w0, w1, wroot, bias)


def _final(a0, a1, c0, c1, t1, seq, w0, w1, wroot, bias, wpa, wpb, bp, n_users):
    def body(a0r, a1r, c0r, c1r, t1r, seqr, w0r, w1r, wrr, br, wpar, wpbr, bpr,
             outr):
        m0 = a0r[:] / jnp.maximum(c0r[:, 0:1], 1.0)
        m1 = a1r[:] / jnp.maximum(c1r[:, 0:1], 1.0)
        t2 = (jnp.dot(m0, w0r[:], preferred_element_type=F32)
              + jnp.dot(m1, w1r[:], preferred_element_type=F32)
              + jnp.dot(t1r[:], wrr[:], preferred_element_type=F32)
              + br[:])
        outr[:] = (jnp.dot(t2, wpar[:], preferred_element_type=F32)
                   + jnp.dot(seqr[:], wpbr[:], preferred_element_type=F32)
                   + bpr[:])

    return pl.pallas_call(
        body,
        grid=(N_T // BD,),
        in_specs=[
            pl.BlockSpec((BD, H), lambda i: (i, 0)),
            pl.BlockSpec((BD, H), lambda i: (i, 0)),
            pl.BlockSpec((BD, H), lambda i: (i, 0)),
            pl.BlockSpec((BD, H), lambda i: (i, 0)),
            pl.BlockSpec((BD, H), lambda i: (i, 0)),
            pl.BlockSpec((BD, H), lambda i: (i, 0)),
            pl.BlockSpec((H, H), lambda i: (0, 0)),
            pl.BlockSpec((H, H), lambda i: (0, 0)),
            pl.BlockSpec((H, H), lambda i: (0, 0)),
            pl.BlockSpec((1, H), lambda i: (0, 0)),
            pl.BlockSpec((H, n_users), lambda i: (0, 0)),
            pl.BlockSpec((H, n_users), lambda i: (0, 0)),
            pl.BlockSpec((1, n_users), lambda i: (0, 0)),
        ],
        out_specs=pl.BlockSpec((BD, n_users), lambda i: (i, 0)),
        out_shape=jax.ShapeDtypeStruct((N_T, n_users), F32),
    )(a0, a1, c0, c1, t1, seq, w0, w1, wroot, bias, wpa, wpb, bp)


def kernel(padded_trajs, trajs_len, edge_index, edge_type, loc_emb, traj_emb,
           W_ih, W_hh, b_ih, b_hh, r1_Wrel, r1_Wroot, r1_b,
           r2_Wrel, r2_Wroot, r2_b, W_pred, b_pred):
    padded_trajs = padded_trajs.astype(jnp.int32)
    trajs_len = trajs_len.astype(jnp.int32)
    edge_index = edge_index.astype(jnp.int32)
    edge_type = edge_type.astype(jnp.int32)
    n_users = W_pred.shape[0]

    tok = jnp.pad(padded_trajs, ((0, NP - N_T), (0, 0)))
    idx3 = tok.reshape(NW, GCH, CH)
    xg = _sc_gather(loc_emb, idx3)
    xseq = xg.reshape(NP, L * H)
    lens_b = jnp.broadcast_to(
        jnp.pad(trajs_len, (0, NP - N_T), constant_values=1).reshape(NP, 1),
        (NP, H))
    seq = _lstm(xseq, lens_b, W_ih.T, W_hh.T, (b_ih + b_hh).reshape(1, -1))

    src = jnp.pad(edge_index[0], (0, EPAD - E))
    dst = jnp.pad(edge_index[1], (0, EPAD - E))
    typ = jnp.pad(edge_type, (0, EPAD - E), constant_values=3)
    enc3 = (src | (dst << 14) | (typ << 28)).reshape(16, ECH, CH)
    z128 = jnp.zeros((RPT, H), F32)
    o128 = jnp.ones((CH, H), F32)

    cnt = _sc_counts(enc3, z128, o128)
    agg1 = _sc_segsum(traj_emb, enc3, z128)
    t1 = _dense1(agg1[0, :N_T], agg1[1, :N_T], cnt[0, :N_T], cnt[1, :N_T],
                 traj_emb, r1_Wrel[0], r1_Wrel[1], r1_Wroot,
                 r1_b.reshape(1, -1))
    agg2 = _sc_segsum(t1, enc3, z128)
    wp = W_pred.T
    logits = _final(agg2[0, :N_T], agg2[1, :N_T], cnt[0, :N_T], cnt[1, :N_T],
                    t1, seq[:N_T], r2_Wrel[0], r2_Wrel[1], r2_Wroot,
                    r2_b.reshape(1, -1), wp[:H], wp[H:],
                    b_pred.reshape(1, -1), n_users)
    return logits
